# baseline (speedup 1.0000x reference)
import functools

import jax
import jax.numpy as jnp
from jax.experimental import pallas as pl

B, N, E = 8, 1024, 8192


def _dense1_body(x_ref, w_ref, b_ref, o_ref):
    o_ref[...] = (
        jnp.dot(x_ref[...], w_ref[...], preferred_element_type=jnp.float32)
        + b_ref[...]
    )


def _dense1(x_bf, wT_bf, brow, block_rows=1024):
    M, K = x_bf.shape
    _, Nc = wT_bf.shape
    return pl.pallas_call(
        _dense1_body,
        grid=(M // block_rows,),
        in_specs=[
            pl.BlockSpec((block_rows, K), lambda i: (i, 0)),
            pl.BlockSpec((K, Nc), lambda i: (0, 0)),
            pl.BlockSpec((1, Nc), lambda i: (0, 0)),
        ],
        out_specs=pl.BlockSpec((block_rows, Nc), lambda i: (i, 0)),
        out_shape=jax.ShapeDtypeStruct((M, Nc), jnp.float32),
    )(x_bf, wT_bf, brow)


def _edge_body(xl_ref, xr_ref, src_ref, dstc_ref, dstr_ref, ea_ref, we_ref,
               att_ref, num_ref, den_ref, *, H, C, Ec):
    j = pl.program_id(1)
    src = src_ref[0]
    dstc = dstc_ref[0]
    dstr = dstr_ref[0]
    it_l = jax.lax.broadcasted_iota(jnp.int32, (Ec, N), 1)
    Ss = (src == it_l).astype(jnp.bfloat16)
    Sd = (dstc == it_l).astype(jnp.bfloat16)
    SdT = (jax.lax.broadcasted_iota(jnp.int32, (N, Ec), 0) == dstr)
    xl = xl_ref[0]
    xr = xr_ref[0]
    Gl = jnp.dot(Ss, xl, preferred_element_type=jnp.float32)
    Gr = jnp.dot(Sd, xr, preferred_element_type=jnp.float32)
    ea = ea_ref[0]
    ee = (ea[:, 0:1] * we_ref[0:1, :]
          + ea[:, 1:2] * we_ref[1:2, :]
          + ea[:, 2:3] * we_ref[2:3, :])
    z = Gl + Gr + ee
    m = jnp.where(z >= 0.0, z, 0.2 * z)
    t = m * att_ref[...]
    exb = []
    for h in range(H):
        lh = jnp.sum(t[:, h * C:(h + 1) * C], axis=1, keepdims=True)
        exb.append(jnp.exp(lh).astype(jnp.bfloat16))
    ex4 = jnp.concatenate(exb, axis=1).astype(jnp.float32)
    exw = jnp.concatenate(
        [jnp.broadcast_to(e.astype(jnp.float32), (Ec, C)) for e in exb],
        axis=1)
    Wn = (Gl * exw).astype(jnp.bfloat16)
    numc = jnp.dot(SdT.astype(jnp.bfloat16), Wn,
                   preferred_element_type=jnp.float32)
    denc = jnp.dot(SdT.astype(jnp.float32), ex4,
                   preferred_element_type=jnp.float32)

    @pl.when(j == 0)
    def _():
        num_ref[0] = numc
        den_ref[0] = denc

    @pl.when(j > 0)
    def _():
        num_ref[0] += numc
        den_ref[0] += denc


def _edge_stage(xl_b, xr_b, src_col, dst_col, dst_row, ea_c, weT, att_row,
                H, C, Ec):
    D = H * C
    nj = E // Ec
    body = functools.partial(_edge_body, H=H, C=C, Ec=Ec)
    num, den = pl.pallas_call(
        body,
        grid=(B, nj),
        in_specs=[
            pl.BlockSpec((1, N, D), lambda b, j: (b, 0, 0)),
            pl.BlockSpec((1, N, D), lambda b, j: (b, 0, 0)),
            pl.BlockSpec((1, Ec, 1), lambda b, j, nj=nj: (b * nj + j, 0, 0)),
            pl.BlockSpec((1, Ec, 1), lambda b, j, nj=nj: (b * nj + j, 0, 0)),
            pl.BlockSpec((1, 1, Ec), lambda b, j, nj=nj: (b * nj + j, 0, 0)),
            pl.BlockSpec((1, Ec, 3), lambda b, j, nj=nj: (b * nj + j, 0, 0)),
            pl.BlockSpec((8, D), lambda b, j: (0, 0)),
            pl.BlockSpec((1, D), lambda b, j: (0, 0)),
        ],
        out_specs=[
            pl.BlockSpec((1, N, D), lambda b, j: (b, 0, 0)),
            pl.BlockSpec((1, N, 4), lambda b, j: (b, 0, 0)),
        ],
        out_shape=[
            jax.ShapeDtypeStruct((B, N, D), jnp.float32),
            jax.ShapeDtypeStruct((B, N, 4), jnp.float32),
        ],
    )(xl_b, xr_b, src_col, dst_col, dst_row, ea_c, weT, att_row)
    return num, den


def _dense2_body(num_ref, den_ref, b1_ref, w_ref, b_ref, o_ref, *, C1):
    num = num_ref[...]
    den = den_ref[...]
    parts = [num[:, h * C1:(h + 1) * C1] / (den[:, h:h + 1] + 1e-16)
             for h in range(4)]
    v = jnp.concatenate(parts, axis=1) + b1_ref[...]
    x1 = v * jax.nn.sigmoid(v)
    o_ref[...] = (
        jnp.dot(x1.astype(jnp.bfloat16), w_ref[...],
                preferred_element_type=jnp.float32)
        + b_ref[...]
    )


def _dense2(num1, den1, bias1_row, wT_bf, brow, block_rows=1024):
    M = num1.shape[0]
    K = num1.shape[1]
    _, Nc = wT_bf.shape
    return pl.pallas_call(
        functools.partial(_dense2_body, C1=K // 4),
        grid=(M // block_rows,),
        in_specs=[
            pl.BlockSpec((block_rows, K), lambda i: (i, 0)),
            pl.BlockSpec((block_rows, 4), lambda i: (i, 0)),
            pl.BlockSpec((1, K), lambda i: (0, 0)),
            pl.BlockSpec((K, Nc), lambda i: (0, 0)),
            pl.BlockSpec((1, Nc), lambda i: (0, 0)),
        ],
        out_specs=pl.BlockSpec((block_rows, Nc), lambda i: (i, 0)),
        out_shape=jax.ShapeDtypeStruct((M, Nc), jnp.float32),
    )(num1, den1, bias1_row, wT_bf, brow)


def _head_body(num_ref, den_ref, b2_ref, h_ref, pw_ref, pb_ref,
               wih_ref, whh_ref, bih_ref, bhh_ref,
               rw1_ref, rb1_ref, rw2_ref, rb2_ref,
               ww1_ref, wb1_ref, ww2_ref, wb2_ref,
               aw1_ref, ab1_ref, aw2_ref, ab2_ref,
               dw1_ref, db1_ref, dw2_ref, db2_ref,
               hn_ref, r_ref, w_ref, ap_ref, ad_ref, *, R):
    i = pl.program_id(0)
    num = num_ref[...]
    den = den_ref[...]
    parts = [num[:, h * 256:(h + 1) * 256] / (den[:, h:h + 1] + 1e-16)
             for h in range(4)]
    out2 = jnp.concatenate(parts, axis=1) + b2_ref[...]
    x2 = (jnp.dot(out2.astype(jnp.bfloat16), pw_ref[...],
                  preferred_element_type=jnp.float32) + pb_ref[...])
    hprev = h_ref[...]
    gi = (jnp.dot(x2.astype(jnp.bfloat16), wih_ref[...],
                  preferred_element_type=jnp.float32) + bih_ref[...])
    gh = (jnp.dot(hprev.astype(jnp.bfloat16), whh_ref[...],
                  preferred_element_type=jnp.float32) + bhh_ref[...])
    r_g = jax.nn.sigmoid(gi[:, 0:256] + gh[:, 0:256])
    z_g = jax.nn.sigmoid(gi[:, 256:512] + gh[:, 256:512])
    n_g = jnp.tanh(gi[:, 512:768] + r_g * gh[:, 512:768])
    hn = (1.0 - z_g) * n_g + z_g * hprev
    hn_ref[...] = hn
    hb = hn.astype(jnp.bfloat16)

    def mlp(w1, b1, w2, b2):
        y = (jnp.dot(hb, w1[...], preferred_element_type=jnp.float32)
             + b1[...])
        y = y * jax.nn.sigmoid(y)
        return (jnp.dot(y.astype(jnp.bfloat16), w2[...],
                        preferred_element_type=jnp.float32) + b2[...])

    r_ref[...] = mlp(rw1_ref, rb1_ref, rw2_ref, rb2_ref)
    w_ref[...] = jax.nn.sigmoid(mlp(ww1_ref, wb1_ref, ww2_ref, wb2_ref)) * 0.95 + 0.05
    ap = jax.nn.sigmoid(mlp(aw1_ref, ab1_ref, aw2_ref, ab2_ref))
    part = jnp.sum(ap) * (1.0 / N)

    @pl.when(i % 2 == 0)
    def _():
        ap_ref[...] = jnp.full((1, 1, 128), part, jnp.float32)

    @pl.when(i % 2 == 1)
    def _():
        ap_ref[...] += part + 0.0001

    ad_ref[...] = jax.nn.sigmoid(mlp(dw1_ref, db1_ref, dw2_ref, db2_ref)) + 0.0001


def _heads(num2, den2, bias2_row, h_flat, wd, R=512):
    M = num2.shape[0]
    grid = (M // R,)
    full = lambda shape: pl.BlockSpec(shape, lambda i: tuple(0 for _ in shape))
    outs = pl.pallas_call(
        functools.partial(_head_body, R=R),
        grid=grid,
        in_specs=[
            pl.BlockSpec((R, 1024), lambda i: (i, 0)),
            pl.BlockSpec((R, 4), lambda i: (i, 0)),
            full((1, 1024)),
            pl.BlockSpec((R, 256), lambda i: (i, 0)),
            full((1024, 256)), full((1, 256)),
            full((256, 768)), full((256, 768)), full((1, 768)), full((1, 768)),
            full((256, 256)), full((1, 256)), full((256, 2)), full((1, 2)),
            full((256, 256)), full((1, 256)), full((256, 2)), full((1, 2)),
            full((256, 128)), full((1, 128)), full((128, 1)), full((1, 1)),
            full((256, 128)), full((1, 128)), full((128, 1)), full((1, 1)),
        ],
        out_specs=[
            pl.BlockSpec((R, 256), lambda i: (i, 0)),
            pl.BlockSpec((R, 2), lambda i: (i, 0)),
            pl.BlockSpec((R, 2), lambda i: (i, 0)),
            pl.BlockSpec((1, 1, 128), lambda i: (i // 2, 0, 0)),
            pl.BlockSpec((R, 1), lambda i: (i, 0)),
        ],
        out_shape=[
            jax.ShapeDtypeStruct((M, 256), jnp.float32),
            jax.ShapeDtypeStruct((M, 2), jnp.float32),
            jax.ShapeDtypeStruct((M, 2), jnp.float32),
            jax.ShapeDtypeStruct((B, 1, 128), jnp.float32),
            jax.ShapeDtypeStruct((M, 1), jnp.float32),
        ],
    )(num2, den2, bias2_row, h_flat, *wd)
    return outs


def kernel(h, e_proj, f_Lt, edges, edge_attr, params):
    p = params
    bf = jnp.bfloat16
    x = jnp.concatenate([e_proj, f_Lt], axis=-1)
    x_flat = x.reshape(B * N, 258)

    src = edges[:, 0, :].astype(jnp.int32)
    dst = edges[:, 1, :].astype(jnp.int32)
    ea = edge_attr.reshape(B * E, 3)

    w1 = jnp.concatenate([p['c1_Wl'], p['c1_Wr']], axis=0)
    b1 = jnp.concatenate([p['c1_bl'], p['c1_br']])[None, :]
    y1 = _dense1(x_flat.astype(bf), w1.T.astype(bf), b1)
    xl1 = y1[:, :256].astype(bf).reshape(B, N, 256)
    xr1 = y1[:, 256:].astype(bf).reshape(B, N, 256)

    Ec1 = 1024
    nj1 = E // Ec1
    src_c1 = src.reshape(B * nj1, Ec1, 1)
    dst_c1 = dst.reshape(B * nj1, Ec1, 1)
    dst_r1 = dst.reshape(B * nj1, 1, Ec1)
    ea_c1 = ea.reshape(B * nj1, Ec1, 3)
    we1 = jnp.zeros((8, 256), jnp.float32).at[:3].set(p['c1_We'].T)
    att1 = p['c1_att'].reshape(1, 256)
    num1, den1 = _edge_stage(xl1, xr1, src_c1, dst_c1, dst_r1, ea_c1,
                             we1, att1, H=4, C=64, Ec=Ec1)

    w2 = jnp.concatenate([p['c2_Wl'], p['c2_Wr']], axis=0)
    b2 = jnp.concatenate([p['c2_bl'], p['c2_br']])[None, :]
    y2 = _dense2(num1.reshape(B * N, 256), den1.reshape(B * N, 4),
                 p['c1_bias'][None, :], w2.T.astype(bf), b2)
    xl2 = y2[:, :1024].astype(bf).reshape(B, N, 1024)
    xr2 = y2[:, 1024:].astype(bf).reshape(B, N, 1024)

    Ec2 = 2048
    nj2 = E // Ec2
    src_c2 = src.reshape(B * nj2, Ec2, 1)
    dst_c2 = dst.reshape(B * nj2, Ec2, 1)
    dst_r2 = dst.reshape(B * nj2, 1, Ec2)
    ea_c2 = ea.reshape(B * nj2, Ec2, 3)
    we2 = jnp.zeros((8, 1024), jnp.float32).at[:3].set(p['c2_We'].T)
    att2 = p['c2_att'].reshape(1, 1024)
    num2, den2 = _edge_stage(xl2, xr2, src_c2, dst_c2, dst_r2, ea_c2,
                             we2, att2, H=4, C=256, Ec=Ec2)

    wd = (
        p['proj_W'].T.astype(bf), p['proj_b'][None, :],
        p['gru_Wih'].T.astype(bf), p['gru_Whh'].T.astype(bf),
        p['gru_bih'][None, :], p['gru_bhh'][None, :],
        p['res_W1'].T.astype(bf), p['res_b1'][None, :],
        p['res_W2'].T.astype(bf), p['res_b2'][None, :],
        p['wh_W1'].T.astype(bf), p['wh_b1'][None, :],
        p['wh_W2'].T.astype(bf), p['wh_b2'][None, :],
        p['ap_W1'].T.astype(bf), p['ap_b1'][None, :],
        p['ap_W2'].T.astype(bf), p['ap_b2'][None, :],
        p['ad_W1'].T.astype(bf), p['ad_b1'][None, :],
        p['ad_W2'].T.astype(bf), p['ad_b2'][None, :],
    )
    hn, r, w, ap, ad = _heads(num2.reshape(B * N, 1024),
                              den2.reshape(B * N, 4),
                              p['c2_bias'][None, :],
                              h.reshape(B * N, 256), wd)
    h_new = hn.reshape(B, N, 256)
    return (h_new,
            r.reshape(B, N, 2),
            w.reshape(B, N, 2),
            ap[:, 0, 0:1],
            ad.reshape(B, N, 1))


# speedup vs baseline: 1.0775x; 1.0775x over previous
import functools

import jax
import jax.numpy as jnp
from jax.experimental import pallas as pl

B, N, E = 8, 1024, 8192


def _dense1_body(x_ref, w_ref, b_ref, o_ref):
    o_ref[...] = (
        jnp.dot(x_ref[...], w_ref[...], preferred_element_type=jnp.float32)
        + b_ref[...]
    )


def _dense1(x_bf, wT_bf, brow, block_rows=1024):
    M, K = x_bf.shape
    _, Nc = wT_bf.shape
    return pl.pallas_call(
        _dense1_body,
        grid=(M // block_rows,),
        in_specs=[
            pl.BlockSpec((block_rows, K), lambda i: (i, 0)),
            pl.BlockSpec((K, Nc), lambda i: (0, 0)),
            pl.BlockSpec((1, Nc), lambda i: (0, 0)),
        ],
        out_specs=pl.BlockSpec((block_rows, Nc), lambda i: (i, 0)),
        out_shape=jax.ShapeDtypeStruct((M, Nc), jnp.float32),
    )(x_bf, wT_bf, brow)


def _edge_body(xl_ref, xr_ref, src_ref, dstc_ref, dstr_ref, ea_ref, we_ref,
               att_ref, num_ref, den_ref, *, H, C, Ec):
    j = pl.program_id(1)
    src = src_ref[0]
    dstc = dstc_ref[0]
    dstr = dstr_ref[0]
    it_l = jax.lax.broadcasted_iota(jnp.int32, (Ec, N), 1)
    Ss = (src == it_l).astype(jnp.bfloat16)
    Sd = (dstc == it_l).astype(jnp.bfloat16)
    SdT = (jax.lax.broadcasted_iota(jnp.int32, (N, Ec), 0) == dstr)
    xl = xl_ref[0]
    xr = xr_ref[0]
    Gl = jnp.dot(Ss, xl, preferred_element_type=jnp.float32)
    Gr = jnp.dot(Sd, xr, preferred_element_type=jnp.float32)
    ea = ea_ref[0]
    ee = (ea[:, 0:1] * we_ref[0:1, :]
          + ea[:, 1:2] * we_ref[1:2, :]
          + ea[:, 2:3] * we_ref[2:3, :])
    z = Gl + Gr + ee
    m = jnp.where(z >= 0.0, z, 0.2 * z)
    t = m * att_ref[...]
    exb = []
    for h in range(H):
        lh = jnp.sum(t[:, h * C:(h + 1) * C], axis=1, keepdims=True)
        exb.append(jnp.exp(lh).astype(jnp.bfloat16))
    ex4 = jnp.concatenate(exb, axis=1).astype(jnp.float32)
    exw = jnp.concatenate(
        [jnp.broadcast_to(e.astype(jnp.float32), (Ec, C)) for e in exb],
        axis=1)
    Wn = (Gl * exw).astype(jnp.bfloat16)
    SdTb = SdT.astype(jnp.bfloat16)
    numc = jnp.dot(SdTb, Wn, preferred_element_type=jnp.float32)
    denc = jnp.dot(SdTb, ex4.astype(jnp.bfloat16),
                   preferred_element_type=jnp.float32)

    @pl.when(j == 0)
    def _():
        num_ref[0] = numc
        den_ref[0] = denc

    @pl.when(j > 0)
    def _():
        num_ref[0] += numc
        den_ref[0] += denc


def _edge_stage(xl_b, xr_b, src_col, dst_col, dst_row, ea_c, weT, att_row,
                H, C, Ec):
    D = H * C
    nj = E // Ec
    body = functools.partial(_edge_body, H=H, C=C, Ec=Ec)
    num, den = pl.pallas_call(
        body,
        grid=(B, nj),
        in_specs=[
            pl.BlockSpec((1, N, D), lambda b, j: (b, 0, 0)),
            pl.BlockSpec((1, N, D), lambda b, j: (b, 0, 0)),
            pl.BlockSpec((1, Ec, 1), lambda b, j, nj=nj: (b * nj + j, 0, 0)),
            pl.BlockSpec((1, Ec, 1), lambda b, j, nj=nj: (b * nj + j, 0, 0)),
            pl.BlockSpec((1, 1, Ec), lambda b, j, nj=nj: (b * nj + j, 0, 0)),
            pl.BlockSpec((1, Ec, 3), lambda b, j, nj=nj: (b * nj + j, 0, 0)),
            pl.BlockSpec((8, D), lambda b, j: (0, 0)),
            pl.BlockSpec((1, D), lambda b, j: (0, 0)),
        ],
        out_specs=[
            pl.BlockSpec((1, N, D), lambda b, j: (b, 0, 0)),
            pl.BlockSpec((1, N, 4), lambda b, j: (b, 0, 0)),
        ],
        out_shape=[
            jax.ShapeDtypeStruct((B, N, D), jnp.float32),
            jax.ShapeDtypeStruct((B, N, 4), jnp.float32),
        ],
    )(xl_b, xr_b, src_col, dst_col, dst_row, ea_c, weT, att_row)
    return num, den


def _dense2_body(num_ref, den_ref, b1_ref, w_ref, b_ref, o_ref, *, C1):
    num = num_ref[...]
    den = den_ref[...]
    parts = [num[:, h * C1:(h + 1) * C1] / (den[:, h:h + 1] + 1e-16)
             for h in range(4)]
    v = jnp.concatenate(parts, axis=1) + b1_ref[...]
    x1 = v * jax.nn.sigmoid(v)
    o_ref[...] = (
        jnp.dot(x1.astype(jnp.bfloat16), w_ref[...],
                preferred_element_type=jnp.float32)
        + b_ref[...]
    )


def _dense2(num1, den1, bias1_row, wT_bf, brow, block_rows=1024):
    M = num1.shape[0]
    K = num1.shape[1]
    _, Nc = wT_bf.shape
    return pl.pallas_call(
        functools.partial(_dense2_body, C1=K // 4),
        grid=(M // block_rows,),
        in_specs=[
            pl.BlockSpec((block_rows, K), lambda i: (i, 0)),
            pl.BlockSpec((block_rows, 4), lambda i: (i, 0)),
            pl.BlockSpec((1, K), lambda i: (0, 0)),
            pl.BlockSpec((K, Nc), lambda i: (0, 0)),
            pl.BlockSpec((1, Nc), lambda i: (0, 0)),
        ],
        out_specs=pl.BlockSpec((block_rows, Nc), lambda i: (i, 0)),
        out_shape=jax.ShapeDtypeStruct((M, Nc), jnp.float32),
    )(num1, den1, bias1_row, wT_bf, brow)


def _head_body(num_ref, den_ref, b2_ref, h_ref, pw_ref, pb_ref,
               wih_ref, whh_ref, bih_ref, bhh_ref,
               rw1_ref, rb1_ref, rw2_ref, rb2_ref,
               ww1_ref, wb1_ref, ww2_ref, wb2_ref,
               aw1_ref, ab1_ref, aw2_ref, ab2_ref,
               dw1_ref, db1_ref, dw2_ref, db2_ref,
               hn_ref, r_ref, w_ref, ap_ref, ad_ref, *, R):
    i = pl.program_id(0)
    num = num_ref[...]
    den = den_ref[...]
    parts = [num[:, h * 256:(h + 1) * 256] / (den[:, h:h + 1] + 1e-16)
             for h in range(4)]
    out2 = jnp.concatenate(parts, axis=1) + b2_ref[...]
    x2 = (jnp.dot(out2.astype(jnp.bfloat16), pw_ref[...],
                  preferred_element_type=jnp.float32) + pb_ref[...])
    hprev = h_ref[...]
    gi = (jnp.dot(x2.astype(jnp.bfloat16), wih_ref[...],
                  preferred_element_type=jnp.float32) + bih_ref[...])
    gh = (jnp.dot(hprev.astype(jnp.bfloat16), whh_ref[...],
                  preferred_element_type=jnp.float32) + bhh_ref[...])
    r_g = jax.nn.sigmoid(gi[:, 0:256] + gh[:, 0:256])
    z_g = jax.nn.sigmoid(gi[:, 256:512] + gh[:, 256:512])
    n_g = jnp.tanh(gi[:, 512:768] + r_g * gh[:, 512:768])
    hn = (1.0 - z_g) * n_g + z_g * hprev
    hn_ref[...] = hn
    hb = hn.astype(jnp.bfloat16)

    def mlp(w1, b1, w2, b2):
        y = (jnp.dot(hb, w1[...], preferred_element_type=jnp.float32)
             + b1[...])
        y = y * jax.nn.sigmoid(y)
        return (jnp.dot(y.astype(jnp.bfloat16), w2[...],
                        preferred_element_type=jnp.float32) + b2[...])

    r_ref[...] = mlp(rw1_ref, rb1_ref, rw2_ref, rb2_ref)
    w_ref[...] = jax.nn.sigmoid(mlp(ww1_ref, wb1_ref, ww2_ref, wb2_ref)) * 0.95 + 0.05
    ap = jax.nn.sigmoid(mlp(aw1_ref, ab1_ref, aw2_ref, ab2_ref))
    part = jnp.sum(ap) * (1.0 / N)

    @pl.when(i % 2 == 0)
    def _():
        ap_ref[...] = jnp.full((1, 1, 128), part, jnp.float32)

    @pl.when(i % 2 == 1)
    def _():
        ap_ref[...] += part + 0.0001

    ad_ref[...] = jax.nn.sigmoid(mlp(dw1_ref, db1_ref, dw2_ref, db2_ref)) + 0.0001


def _heads(num2, den2, bias2_row, h_flat, wd, R=512):
    M = num2.shape[0]
    grid = (M // R,)
    full = lambda shape: pl.BlockSpec(shape, lambda i: tuple(0 for _ in shape))
    outs = pl.pallas_call(
        functools.partial(_head_body, R=R),
        grid=grid,
        in_specs=[
            pl.BlockSpec((R, 1024), lambda i: (i, 0)),
            pl.BlockSpec((R, 4), lambda i: (i, 0)),
            full((1, 1024)),
            pl.BlockSpec((R, 256), lambda i: (i, 0)),
            full((1024, 256)), full((1, 256)),
            full((256, 768)), full((256, 768)), full((1, 768)), full((1, 768)),
            full((256, 256)), full((1, 256)), full((256, 2)), full((1, 2)),
            full((256, 256)), full((1, 256)), full((256, 2)), full((1, 2)),
            full((256, 128)), full((1, 128)), full((128, 1)), full((1, 1)),
            full((256, 128)), full((1, 128)), full((128, 1)), full((1, 1)),
        ],
        out_specs=[
            pl.BlockSpec((R, 256), lambda i: (i, 0)),
            pl.BlockSpec((R, 2), lambda i: (i, 0)),
            pl.BlockSpec((R, 2), lambda i: (i, 0)),
            pl.BlockSpec((1, 1, 128), lambda i: (i // 2, 0, 0)),
            pl.BlockSpec((R, 1), lambda i: (i, 0)),
        ],
        out_shape=[
            jax.ShapeDtypeStruct((M, 256), jnp.float32),
            jax.ShapeDtypeStruct((M, 2), jnp.float32),
            jax.ShapeDtypeStruct((M, 2), jnp.float32),
            jax.ShapeDtypeStruct((B, 1, 128), jnp.float32),
            jax.ShapeDtypeStruct((M, 1), jnp.float32),
        ],
    )(num2, den2, bias2_row, h_flat, *wd)
    return outs


def kernel(h, e_proj, f_Lt, edges, edge_attr, params):
    p = params
    bf = jnp.bfloat16
    x = jnp.concatenate([e_proj, f_Lt], axis=-1)
    x_flat = x.reshape(B * N, 258)

    src = edges[:, 0, :].astype(jnp.int32)
    dst = edges[:, 1, :].astype(jnp.int32)
    ea = edge_attr.reshape(B * E, 3)

    w1 = jnp.concatenate([p['c1_Wl'], p['c1_Wr']], axis=0)
    b1 = jnp.concatenate([p['c1_bl'], p['c1_br']])[None, :]
    y1 = _dense1(x_flat.astype(bf), w1.T.astype(bf), b1)
    xl1 = y1[:, :256].astype(bf).reshape(B, N, 256)
    xr1 = y1[:, 256:].astype(bf).reshape(B, N, 256)

    Ec1 = 2048
    nj1 = E // Ec1
    src_c1 = src.reshape(B * nj1, Ec1, 1)
    dst_c1 = dst.reshape(B * nj1, Ec1, 1)
    dst_r1 = dst.reshape(B * nj1, 1, Ec1)
    ea_c1 = ea.reshape(B * nj1, Ec1, 3)
    we1 = jnp.zeros((8, 256), jnp.float32).at[:3].set(p['c1_We'].T)
    att1 = p['c1_att'].reshape(1, 256)
    num1, den1 = _edge_stage(xl1, xr1, src_c1, dst_c1, dst_r1, ea_c1,
                             we1, att1, H=4, C=64, Ec=Ec1)

    w2 = jnp.concatenate([p['c2_Wl'], p['c2_Wr']], axis=0)
    b2 = jnp.concatenate([p['c2_bl'], p['c2_br']])[None, :]
    y2 = _dense2(num1.reshape(B * N, 256), den1.reshape(B * N, 4),
                 p['c1_bias'][None, :], w2.T.astype(bf), b2)
    xl2 = y2[:, :1024].astype(bf).reshape(B, N, 1024)
    xr2 = y2[:, 1024:].astype(bf).reshape(B, N, 1024)

    Ec2 = 2048
    nj2 = E // Ec2
    src_c2 = src.reshape(B * nj2, Ec2, 1)
    dst_c2 = dst.reshape(B * nj2, Ec2, 1)
    dst_r2 = dst.reshape(B * nj2, 1, Ec2)
    ea_c2 = ea.reshape(B * nj2, Ec2, 3)
    we2 = jnp.zeros((8, 1024), jnp.float32).at[:3].set(p['c2_We'].T)
    att2 = p['c2_att'].reshape(1, 1024)
    num2, den2 = _edge_stage(xl2, xr2, src_c2, dst_c2, dst_r2, ea_c2,
                             we2, att2, H=4, C=256, Ec=Ec2)

    wd = (
        p['proj_W'].T.astype(bf), p['proj_b'][None, :],
        p['gru_Wih'].T.astype(bf), p['gru_Whh'].T.astype(bf),
        p['gru_bih'][None, :], p['gru_bhh'][None, :],
        p['res_W1'].T.astype(bf), p['res_b1'][None, :],
        p['res_W2'].T.astype(bf), p['res_b2'][None, :],
        p['wh_W1'].T.astype(bf), p['wh_b1'][None, :],
        p['wh_W2'].T.astype(bf), p['wh_b2'][None, :],
        p['ap_W1'].T.astype(bf), p['ap_b1'][None, :],
        p['ap_W2'].T.astype(bf), p['ap_b2'][None, :],
        p['ad_W1'].T.astype(bf), p['ad_b1'][None, :],
        p['ad_W2'].T.astype(bf), p['ad_b2'][None, :],
    )
    hn, r, w, ap, ad = _heads(num2.reshape(B * N, 1024),
                              den2.reshape(B * N, 4),
                              p['c2_bias'][None, :],
                              h.reshape(B * N, 256), wd)
    h_new = hn.reshape(B, N, 256)
    return (h_new,
            r.reshape(B, N, 2),
            w.reshape(B, N, 2),
            ap[:, 0, 0:1],
            ad.reshape(B, N, 1))


# speedup vs baseline: 1.1591x; 1.0757x over previous
import functools

import jax
import jax.numpy as jnp
from jax.experimental import pallas as pl

B, N, E = 8, 1024, 8192


def _dense1_body(x_ref, w_ref, b_ref, ol_ref, or_ref):
    y = (jnp.dot(x_ref[...], w_ref[...], preferred_element_type=jnp.float32)
         + b_ref[...])
    half = y.shape[1] // 2
    ol_ref[...] = y[:, :half].astype(jnp.bfloat16)
    or_ref[...] = y[:, half:].astype(jnp.bfloat16)


def _dense1(x_bf, wT_bf, brow, block_rows=1024):
    M, K = x_bf.shape
    _, Nc = wT_bf.shape
    Nh = Nc // 2
    return pl.pallas_call(
        _dense1_body,
        grid=(M // block_rows,),
        in_specs=[
            pl.BlockSpec((block_rows, K), lambda i: (i, 0)),
            pl.BlockSpec((K, Nc), lambda i: (0, 0)),
            pl.BlockSpec((1, Nc), lambda i: (0, 0)),
        ],
        out_specs=[
            pl.BlockSpec((block_rows, Nh), lambda i: (i, 0)),
            pl.BlockSpec((block_rows, Nh), lambda i: (i, 0)),
        ],
        out_shape=[
            jax.ShapeDtypeStruct((M, Nh), jnp.bfloat16),
            jax.ShapeDtypeStruct((M, Nh), jnp.bfloat16),
        ],
    )(x_bf, wT_bf, brow)


def _edge_body(xl_ref, xr_ref, src_ref, dstc_ref, dstr_ref, ea_ref, we_ref,
               att_ref, num_ref, den_ref, *, H, C, Ec):
    j = pl.program_id(1)
    src = src_ref[0]
    dstc = dstc_ref[0]
    dstr = dstr_ref[0]
    it_l = jax.lax.broadcasted_iota(jnp.int32, (Ec, N), 1)
    Ss = (src == it_l).astype(jnp.bfloat16)
    Sd = (dstc == it_l).astype(jnp.bfloat16)
    SdT = (jax.lax.broadcasted_iota(jnp.int32, (N, Ec), 0) == dstr)
    xl = xl_ref[0]
    xr = xr_ref[0]
    Gl = jnp.dot(Ss, xl, preferred_element_type=jnp.float32)
    Gr = jnp.dot(Sd, xr, preferred_element_type=jnp.float32)
    ea = ea_ref[0]
    ee = (ea[:, 0:1] * we_ref[0:1, :]
          + ea[:, 1:2] * we_ref[1:2, :]
          + ea[:, 2:3] * we_ref[2:3, :])
    z = Gl + Gr + ee
    m = jnp.where(z >= 0.0, z, 0.2 * z)
    t = m * att_ref[...]
    exb = []
    for h in range(H):
        lh = jnp.sum(t[:, h * C:(h + 1) * C], axis=1, keepdims=True)
        exb.append(jnp.exp(lh).astype(jnp.bfloat16))
    ex4 = jnp.concatenate(exb, axis=1).astype(jnp.float32)
    exw = jnp.concatenate(
        [jnp.broadcast_to(e.astype(jnp.float32), (Ec, C)) for e in exb],
        axis=1)
    Wn = (Gl * exw).astype(jnp.bfloat16)
    numc = jnp.dot(SdT.astype(jnp.bfloat16), Wn,
                   preferred_element_type=jnp.float32)
    denc = jnp.dot(SdT.astype(jnp.float32), ex4,
                   preferred_element_type=jnp.float32)

    @pl.when(j == 0)
    def _():
        num_ref[0] = numc
        den_ref[0] = denc

    @pl.when(j > 0)
    def _():
        num_ref[0] += numc
        den_ref[0] += denc


def _edge_stage(xl_b, xr_b, src_col, dst_col, dst_row, ea_c, weT, att_row,
                H, C, Ec):
    D = H * C
    nj = E // Ec
    body = functools.partial(_edge_body, H=H, C=C, Ec=Ec)
    num, den = pl.pallas_call(
        body,
        grid=(B, nj),
        in_specs=[
            pl.BlockSpec((1, N, D), lambda b, j: (b, 0, 0)),
            pl.BlockSpec((1, N, D), lambda b, j: (b, 0, 0)),
            pl.BlockSpec((1, Ec, 1), lambda b, j, nj=nj: (b * nj + j, 0, 0)),
            pl.BlockSpec((1, Ec, 1), lambda b, j, nj=nj: (b * nj + j, 0, 0)),
            pl.BlockSpec((1, 1, Ec), lambda b, j, nj=nj: (b * nj + j, 0, 0)),
            pl.BlockSpec((1, Ec, 3), lambda b, j, nj=nj: (b * nj + j, 0, 0)),
            pl.BlockSpec((8, D), lambda b, j: (0, 0)),
            pl.BlockSpec((1, D), lambda b, j: (0, 0)),
        ],
        out_specs=[
            pl.BlockSpec((1, N, D), lambda b, j: (b, 0, 0)),
            pl.BlockSpec((1, N, 4), lambda b, j: (b, 0, 0)),
        ],
        out_shape=[
            jax.ShapeDtypeStruct((B, N, D), jnp.float32),
            jax.ShapeDtypeStruct((B, N, 4), jnp.float32),
        ],
    )(xl_b, xr_b, src_col, dst_col, dst_row, ea_c, weT, att_row)
    return num, den


def _dense2_body(num_ref, den_ref, b1_ref, w_ref, b_ref, ol_ref, or_ref,
                 *, C1):
    num = num_ref[...]
    den = den_ref[...]
    parts = [num[:, h * C1:(h + 1) * C1] / (den[:, h:h + 1] + 1e-16)
             for h in range(4)]
    v = jnp.concatenate(parts, axis=1) + b1_ref[...]
    x1 = v * jax.nn.sigmoid(v)
    y = (jnp.dot(x1.astype(jnp.bfloat16), w_ref[...],
                 preferred_element_type=jnp.float32)
         + b_ref[...])
    half = y.shape[1] // 2
    ol_ref[...] = y[:, :half].astype(jnp.bfloat16)
    or_ref[...] = y[:, half:].astype(jnp.bfloat16)


def _dense2(num1, den1, bias1_row, wT_bf, brow, block_rows=1024):
    M = num1.shape[0]
    K = num1.shape[1]
    _, Nc = wT_bf.shape
    Nh = Nc // 2
    return pl.pallas_call(
        functools.partial(_dense2_body, C1=K // 4),
        grid=(M // block_rows,),
        in_specs=[
            pl.BlockSpec((block_rows, K), lambda i: (i, 0)),
            pl.BlockSpec((block_rows, 4), lambda i: (i, 0)),
            pl.BlockSpec((1, K), lambda i: (0, 0)),
            pl.BlockSpec((K, Nc), lambda i: (0, 0)),
            pl.BlockSpec((1, Nc), lambda i: (0, 0)),
        ],
        out_specs=[
            pl.BlockSpec((block_rows, Nh), lambda i: (i, 0)),
            pl.BlockSpec((block_rows, Nh), lambda i: (i, 0)),
        ],
        out_shape=[
            jax.ShapeDtypeStruct((M, Nh), jnp.bfloat16),
            jax.ShapeDtypeStruct((M, Nh), jnp.bfloat16),
        ],
    )(num1, den1, bias1_row, wT_bf, brow)


def _head_body(num_ref, den_ref, b2_ref, h_ref, pw_ref, pb_ref,
               wih_ref, whh_ref, bih_ref, bhh_ref,
               rw1_ref, rb1_ref, rw2_ref, rb2_ref,
               ww1_ref, wb1_ref, ww2_ref, wb2_ref,
               aw1_ref, ab1_ref, aw2_ref, ab2_ref,
               dw1_ref, db1_ref, dw2_ref, db2_ref,
               hn_ref, r_ref, w_ref, ap_ref, ad_ref, *, R):
    i = pl.program_id(0)
    num = num_ref[...]
    den = den_ref[...]
    parts = [num[:, h * 256:(h + 1) * 256] / (den[:, h:h + 1] + 1e-16)
             for h in range(4)]
    out2 = jnp.concatenate(parts, axis=1) + b2_ref[...]
    x2 = (jnp.dot(out2.astype(jnp.bfloat16), pw_ref[...],
                  preferred_element_type=jnp.float32) + pb_ref[...])
    hprev = h_ref[...]
    gi = (jnp.dot(x2.astype(jnp.bfloat16), wih_ref[...],
                  preferred_element_type=jnp.float32) + bih_ref[...])
    gh = (jnp.dot(hprev.astype(jnp.bfloat16), whh_ref[...],
                  preferred_element_type=jnp.float32) + bhh_ref[...])
    r_g = jax.nn.sigmoid(gi[:, 0:256] + gh[:, 0:256])
    z_g = jax.nn.sigmoid(gi[:, 256:512] + gh[:, 256:512])
    n_g = jnp.tanh(gi[:, 512:768] + r_g * gh[:, 512:768])
    hn = (1.0 - z_g) * n_g + z_g * hprev
    hn_ref[...] = hn
    hb = hn.astype(jnp.bfloat16)

    def mlp(w1, b1, w2, b2):
        y = (jnp.dot(hb, w1[...], preferred_element_type=jnp.float32)
             + b1[...])
        y = y * jax.nn.sigmoid(y)
        return (jnp.dot(y.astype(jnp.bfloat16), w2[...],
                        preferred_element_type=jnp.float32) + b2[...])

    r_ref[...] = mlp(rw1_ref, rb1_ref, rw2_ref, rb2_ref)
    w_ref[...] = jax.nn.sigmoid(mlp(ww1_ref, wb1_ref, ww2_ref, wb2_ref)) * 0.95 + 0.05
    ap = jax.nn.sigmoid(mlp(aw1_ref, ab1_ref, aw2_ref, ab2_ref))
    part = jnp.sum(ap) * (1.0 / N)

    @pl.when(i % 2 == 0)
    def _():
        ap_ref[...] = jnp.full((1, 1, 128), part, jnp.float32)

    @pl.when(i % 2 == 1)
    def _():
        ap_ref[...] += part + 0.0001

    ad_ref[...] = jax.nn.sigmoid(mlp(dw1_ref, db1_ref, dw2_ref, db2_ref)) + 0.0001


def _heads(num2, den2, bias2_row, h_flat, wd, R=512):
    M = num2.shape[0]
    grid = (M // R,)
    full = lambda shape: pl.BlockSpec(shape, lambda i: tuple(0 for _ in shape))
    outs = pl.pallas_call(
        functools.partial(_head_body, R=R),
        grid=grid,
        in_specs=[
            pl.BlockSpec((R, 1024), lambda i: (i, 0)),
            pl.BlockSpec((R, 4), lambda i: (i, 0)),
            full((1, 1024)),
            pl.BlockSpec((R, 256), lambda i: (i, 0)),
            full((1024, 256)), full((1, 256)),
            full((256, 768)), full((256, 768)), full((1, 768)), full((1, 768)),
            full((256, 256)), full((1, 256)), full((256, 2)), full((1, 2)),
            full((256, 256)), full((1, 256)), full((256, 2)), full((1, 2)),
            full((256, 128)), full((1, 128)), full((128, 1)), full((1, 1)),
            full((256, 128)), full((1, 128)), full((128, 1)), full((1, 1)),
        ],
        out_specs=[
            pl.BlockSpec((R, 256), lambda i: (i, 0)),
            pl.BlockSpec((R, 2), lambda i: (i, 0)),
            pl.BlockSpec((R, 2), lambda i: (i, 0)),
            pl.BlockSpec((1, 1, 128), lambda i: (i // 2, 0, 0)),
            pl.BlockSpec((R, 1), lambda i: (i, 0)),
        ],
        out_shape=[
            jax.ShapeDtypeStruct((M, 256), jnp.float32),
            jax.ShapeDtypeStruct((M, 2), jnp.float32),
            jax.ShapeDtypeStruct((M, 2), jnp.float32),
            jax.ShapeDtypeStruct((B, 1, 128), jnp.float32),
            jax.ShapeDtypeStruct((M, 1), jnp.float32),
        ],
    )(num2, den2, bias2_row, h_flat, *wd)
    return outs


def kernel(h, e_proj, f_Lt, edges, edge_attr, params):
    p = params
    bf = jnp.bfloat16
    x = jnp.concatenate([e_proj, f_Lt], axis=-1)
    x_flat = x.reshape(B * N, 258)

    src = edges[:, 0, :].astype(jnp.int32)
    dst = edges[:, 1, :].astype(jnp.int32)
    ea = edge_attr.reshape(B * E, 3)

    w1 = jnp.concatenate([p['c1_Wl'], p['c1_Wr']], axis=0)
    b1 = jnp.concatenate([p['c1_bl'], p['c1_br']])[None, :]
    xl1, xr1 = _dense1(x_flat.astype(bf), w1.T.astype(bf), b1)
    xl1 = xl1.reshape(B, N, 256)
    xr1 = xr1.reshape(B, N, 256)

    Ec1 = 2048
    nj1 = E // Ec1
    src_c1 = src.reshape(B * nj1, Ec1, 1)
    dst_c1 = dst.reshape(B * nj1, Ec1, 1)
    dst_r1 = dst.reshape(B * nj1, 1, Ec1)
    ea_c1 = ea.reshape(B * nj1, Ec1, 3)
    we1 = jnp.zeros((8, 256), jnp.float32).at[:3].set(p['c1_We'].T)
    att1 = p['c1_att'].reshape(1, 256)
    num1, den1 = _edge_stage(xl1, xr1, src_c1, dst_c1, dst_r1, ea_c1,
                             we1, att1, H=4, C=64, Ec=Ec1)

    w2 = jnp.concatenate([p['c2_Wl'], p['c2_Wr']], axis=0)
    b2 = jnp.concatenate([p['c2_bl'], p['c2_br']])[None, :]
    xl2, xr2 = _dense2(num1.reshape(B * N, 256), den1.reshape(B * N, 4),
                       p['c1_bias'][None, :], w2.T.astype(bf), b2)
    xl2 = xl2.reshape(B, N, 1024)
    xr2 = xr2.reshape(B, N, 1024)

    Ec2 = 2048
    nj2 = E // Ec2
    src_c2 = src.reshape(B * nj2, Ec2, 1)
    dst_c2 = dst.reshape(B * nj2, Ec2, 1)
    dst_r2 = dst.reshape(B * nj2, 1, Ec2)
    ea_c2 = ea.reshape(B * nj2, Ec2, 3)
    we2 = jnp.zeros((8, 1024), jnp.float32).at[:3].set(p['c2_We'].T)
    att2 = p['c2_att'].reshape(1, 1024)
    num2, den2 = _edge_stage(xl2, xr2, src_c2, dst_c2, dst_r2, ea_c2,
                             we2, att2, H=4, C=256, Ec=Ec2)

    wd = (
        p['proj_W'].T.astype(bf), p['proj_b'][None, :],
        p['gru_Wih'].T.astype(bf), p['gru_Whh'].T.astype(bf),
        p['gru_bih'][None, :], p['gru_bhh'][None, :],
        p['res_W1'].T.astype(bf), p['res_b1'][None, :],
        p['res_W2'].T.astype(bf), p['res_b2'][None, :],
        p['wh_W1'].T.astype(bf), p['wh_b1'][None, :],
        p['wh_W2'].T.astype(bf), p['wh_b2'][None, :],
        p['ap_W1'].T.astype(bf), p['ap_b1'][None, :],
        p['ap_W2'].T.astype(bf), p['ap_b2'][None, :],
        p['ad_W1'].T.astype(bf), p['ad_b1'][None, :],
        p['ad_W2'].T.astype(bf), p['ad_b2'][None, :],
    )
    hn, r, w, ap, ad = _heads(num2.reshape(B * N, 1024),
                              den2.reshape(B * N, 4),
                              p['c2_bias'][None, :],
                              h.reshape(B * N, 256), wd)
    h_new = hn.reshape(B, N, 256)
    return (h_new,
            r.reshape(B, N, 2),
            w.reshape(B, N, 2),
            ap[:, 0, 0:1],
            ad.reshape(B, N, 1))


# speedup vs baseline: 1.1686x; 1.0083x over previous
import functools

import jax
import jax.numpy as jnp
from jax.experimental import pallas as pl

B, N, E = 8, 1024, 8192


def _dense1_body(x_ref, w_ref, b_ref, ol_ref, or_ref):
    y = (jnp.dot(x_ref[...].astype(jnp.bfloat16), w_ref[...],
                 preferred_element_type=jnp.float32)
         + b_ref[...])
    half = y.shape[1] // 2
    ol_ref[...] = y[:, :half].astype(jnp.bfloat16)
    or_ref[...] = y[:, half:].astype(jnp.bfloat16)


def _dense1(x_bf, wT_bf, brow, block_rows=1024):
    M, K = x_bf.shape
    _, Nc = wT_bf.shape
    Nh = Nc // 2
    return pl.pallas_call(
        _dense1_body,
        grid=(M // block_rows,),
        in_specs=[
            pl.BlockSpec((block_rows, K), lambda i: (i, 0)),
            pl.BlockSpec((K, Nc), lambda i: (0, 0)),
            pl.BlockSpec((1, Nc), lambda i: (0, 0)),
        ],
        out_specs=[
            pl.BlockSpec((block_rows, Nh), lambda i: (i, 0)),
            pl.BlockSpec((block_rows, Nh), lambda i: (i, 0)),
        ],
        out_shape=[
            jax.ShapeDtypeStruct((M, Nh), jnp.bfloat16),
            jax.ShapeDtypeStruct((M, Nh), jnp.bfloat16),
        ],
    )(x_bf, wT_bf, brow)


def _edge_body(xl_ref, xr_ref, src_ref, dstc_ref, dstr_ref, ea_ref, we_ref,
               att_ref, num_ref, den_ref, *, H, C, Ec):
    j = pl.program_id(1)
    src = src_ref[0]
    dstc = dstc_ref[0]
    dstr = dstr_ref[0]
    it_l = jax.lax.broadcasted_iota(jnp.int32, (Ec, N), 1)
    Ss = (src == it_l).astype(jnp.bfloat16)
    Sd = (dstc == it_l).astype(jnp.bfloat16)
    SdT = (jax.lax.broadcasted_iota(jnp.int32, (N, Ec), 0) == dstr)
    xl = xl_ref[0]
    xr = xr_ref[0]
    Gl = jnp.dot(Ss, xl, preferred_element_type=jnp.float32)
    Gr = jnp.dot(Sd, xr, preferred_element_type=jnp.float32)
    ea = ea_ref[0]
    ee = (ea[:, 0:1] * we_ref[0:1, :]
          + ea[:, 1:2] * we_ref[1:2, :]
          + ea[:, 2:3] * we_ref[2:3, :])
    z = Gl + Gr + ee
    m = jnp.where(z >= 0.0, z, 0.2 * z)
    t = m * att_ref[...]
    exb = []
    for h in range(H):
        lh = jnp.sum(t[:, h * C:(h + 1) * C], axis=1, keepdims=True)
        exb.append(jnp.exp(lh).astype(jnp.bfloat16))
    ex4 = jnp.concatenate(exb, axis=1).astype(jnp.float32)
    exw = jnp.concatenate(
        [jnp.broadcast_to(e.astype(jnp.float32), (Ec, C)) for e in exb],
        axis=1)
    Wn = (Gl * exw).astype(jnp.bfloat16)
    numc = jnp.dot(SdT.astype(jnp.bfloat16), Wn,
                   preferred_element_type=jnp.float32)
    denc = jnp.dot(SdT.astype(jnp.float32), ex4,
                   preferred_element_type=jnp.float32)

    @pl.when(j == 0)
    def _():
        num_ref[0] = numc
        den_ref[0] = denc

    @pl.when(j > 0)
    def _():
        num_ref[0] += numc
        den_ref[0] += denc


def _edge_stage(xl_b, xr_b, src_col, dst_col, dst_row, ea_c, weT, att_row,
                H, C, Ec):
    D = H * C
    nj = E // Ec
    body = functools.partial(_edge_body, H=H, C=C, Ec=Ec)
    num, den = pl.pallas_call(
        body,
        grid=(B, nj),
        in_specs=[
            pl.BlockSpec((1, N, D), lambda b, j: (b, 0, 0)),
            pl.BlockSpec((1, N, D), lambda b, j: (b, 0, 0)),
            pl.BlockSpec((1, Ec, 1), lambda b, j, nj=nj: (b * nj + j, 0, 0)),
            pl.BlockSpec((1, Ec, 1), lambda b, j, nj=nj: (b * nj + j, 0, 0)),
            pl.BlockSpec((1, 1, Ec), lambda b, j, nj=nj: (b * nj + j, 0, 0)),
            pl.BlockSpec((1, Ec, 3), lambda b, j, nj=nj: (b * nj + j, 0, 0)),
            pl.BlockSpec((8, D), lambda b, j: (0, 0)),
            pl.BlockSpec((1, D), lambda b, j: (0, 0)),
        ],
        out_specs=[
            pl.BlockSpec((1, N, D), lambda b, j: (b, 0, 0)),
            pl.BlockSpec((1, N, 4), lambda b, j: (b, 0, 0)),
        ],
        out_shape=[
            jax.ShapeDtypeStruct((B, N, D), jnp.float32),
            jax.ShapeDtypeStruct((B, N, 4), jnp.float32),
        ],
    )(xl_b, xr_b, src_col, dst_col, dst_row, ea_c, weT, att_row)
    return num, den


def _dense2_body(num_ref, den_ref, b1_ref, w_ref, b_ref, ol_ref, or_ref,
                 *, C1):
    num = num_ref[...]
    den = den_ref[...]
    parts = [num[:, h * C1:(h + 1) * C1] / (den[:, h:h + 1] + 1e-16)
             for h in range(4)]
    v = jnp.concatenate(parts, axis=1) + b1_ref[...]
    x1 = v * jax.nn.sigmoid(v)
    y = (jnp.dot(x1.astype(jnp.bfloat16), w_ref[...],
                 preferred_element_type=jnp.float32)
         + b_ref[...])
    half = y.shape[1] // 2
    ol_ref[...] = y[:, :half].astype(jnp.bfloat16)
    or_ref[...] = y[:, half:].astype(jnp.bfloat16)


def _dense2(num1, den1, bias1_row, wT_bf, brow, block_rows=1024):
    M = num1.shape[0]
    K = num1.shape[1]
    _, Nc = wT_bf.shape
    Nh = Nc // 2
    return pl.pallas_call(
        functools.partial(_dense2_body, C1=K // 4),
        grid=(M // block_rows,),
        in_specs=[
            pl.BlockSpec((block_rows, K), lambda i: (i, 0)),
            pl.BlockSpec((block_rows, 4), lambda i: (i, 0)),
            pl.BlockSpec((1, K), lambda i: (0, 0)),
            pl.BlockSpec((K, Nc), lambda i: (0, 0)),
            pl.BlockSpec((1, Nc), lambda i: (0, 0)),
        ],
        out_specs=[
            pl.BlockSpec((block_rows, Nh), lambda i: (i, 0)),
            pl.BlockSpec((block_rows, Nh), lambda i: (i, 0)),
        ],
        out_shape=[
            jax.ShapeDtypeStruct((M, Nh), jnp.bfloat16),
            jax.ShapeDtypeStruct((M, Nh), jnp.bfloat16),
        ],
    )(num1, den1, bias1_row, wT_bf, brow)


def _head_body(num_ref, den_ref, b2_ref, h_ref, pw_ref, pb_ref,
               wih_ref, whh_ref, bih_ref, bhh_ref,
               rw1_ref, rb1_ref, rw2_ref, rb2_ref,
               ww1_ref, wb1_ref, ww2_ref, wb2_ref,
               aw1_ref, ab1_ref, aw2_ref, ab2_ref,
               dw1_ref, db1_ref, dw2_ref, db2_ref,
               hn_ref, r_ref, w_ref, ap_ref, ad_ref, *, R):
    i = pl.program_id(0)
    num = num_ref[...]
    den = den_ref[...]
    parts = [num[:, h * 256:(h + 1) * 256] / (den[:, h:h + 1] + 1e-16)
             for h in range(4)]
    out2 = jnp.concatenate(parts, axis=1) + b2_ref[...]
    x2 = (jnp.dot(out2.astype(jnp.bfloat16), pw_ref[...],
                  preferred_element_type=jnp.float32) + pb_ref[...])
    hprev = h_ref[...]
    gi = (jnp.dot(x2.astype(jnp.bfloat16), wih_ref[...],
                  preferred_element_type=jnp.float32) + bih_ref[...])
    gh = (jnp.dot(hprev.astype(jnp.bfloat16), whh_ref[...],
                  preferred_element_type=jnp.float32) + bhh_ref[...])
    r_g = jax.nn.sigmoid(gi[:, 0:256] + gh[:, 0:256])
    z_g = jax.nn.sigmoid(gi[:, 256:512] + gh[:, 256:512])
    n_g = jnp.tanh(gi[:, 512:768] + r_g * gh[:, 512:768])
    hn = (1.0 - z_g) * n_g + z_g * hprev
    hn_ref[...] = hn
    hb = hn.astype(jnp.bfloat16)

    def mlp(w1, b1, w2, b2):
        y = (jnp.dot(hb, w1[...], preferred_element_type=jnp.float32)
             + b1[...])
        y = y * jax.nn.sigmoid(y)
        return (jnp.dot(y.astype(jnp.bfloat16), w2[...],
                        preferred_element_type=jnp.float32) + b2[...])

    r_ref[...] = mlp(rw1_ref, rb1_ref, rw2_ref, rb2_ref)
    w_ref[...] = jax.nn.sigmoid(mlp(ww1_ref, wb1_ref, ww2_ref, wb2_ref)) * 0.95 + 0.05
    ap = jax.nn.sigmoid(mlp(aw1_ref, ab1_ref, aw2_ref, ab2_ref))
    part = jnp.sum(ap) * (1.0 / N)
    ap_ref[...] = jnp.full((1, 1, 128), part + 0.0001, jnp.float32)
    ad_ref[...] = jax.nn.sigmoid(mlp(dw1_ref, db1_ref, dw2_ref, db2_ref)) + 0.0001


def _heads(num2, den2, bias2_row, h_flat, wd, R=1024):
    M = num2.shape[0]
    grid = (M // R,)
    full = lambda shape: pl.BlockSpec(shape, lambda i: tuple(0 for _ in shape))
    outs = pl.pallas_call(
        functools.partial(_head_body, R=R),
        grid=grid,
        in_specs=[
            pl.BlockSpec((R, 1024), lambda i: (i, 0)),
            pl.BlockSpec((R, 4), lambda i: (i, 0)),
            full((1, 1024)),
            pl.BlockSpec((R, 256), lambda i: (i, 0)),
            full((1024, 256)), full((1, 256)),
            full((256, 768)), full((256, 768)), full((1, 768)), full((1, 768)),
            full((256, 256)), full((1, 256)), full((256, 2)), full((1, 2)),
            full((256, 256)), full((1, 256)), full((256, 2)), full((1, 2)),
            full((256, 128)), full((1, 128)), full((128, 1)), full((1, 1)),
            full((256, 128)), full((1, 128)), full((128, 1)), full((1, 1)),
        ],
        out_specs=[
            pl.BlockSpec((R, 256), lambda i: (i, 0)),
            pl.BlockSpec((R, 2), lambda i: (i, 0)),
            pl.BlockSpec((R, 2), lambda i: (i, 0)),
            pl.BlockSpec((1, 1, 128), lambda i: (i, 0, 0)),
            pl.BlockSpec((R, 1), lambda i: (i, 0)),
        ],
        out_shape=[
            jax.ShapeDtypeStruct((M, 256), jnp.float32),
            jax.ShapeDtypeStruct((M, 2), jnp.float32),
            jax.ShapeDtypeStruct((M, 2), jnp.float32),
            jax.ShapeDtypeStruct((B, 1, 128), jnp.float32),
            jax.ShapeDtypeStruct((M, 1), jnp.float32),
        ],
    )(num2, den2, bias2_row, h_flat, *wd)
    return outs


def kernel(h, e_proj, f_Lt, edges, edge_attr, params):
    p = params
    bf = jnp.bfloat16
    x = jnp.concatenate([e_proj, f_Lt], axis=-1)
    x_flat = x.reshape(B * N, 258)

    src = edges[:, 0, :].astype(jnp.int32)
    dst = edges[:, 1, :].astype(jnp.int32)
    ea = edge_attr.reshape(B * E, 3)

    w1 = jnp.concatenate([p['c1_Wl'], p['c1_Wr']], axis=0)
    b1 = jnp.concatenate([p['c1_bl'], p['c1_br']])[None, :]
    xl1, xr1 = _dense1(x_flat, w1.T.astype(bf), b1)
    xl1 = xl1.reshape(B, N, 256)
    xr1 = xr1.reshape(B, N, 256)

    Ec1 = 2048
    nj1 = E // Ec1
    src_c1 = src.reshape(B * nj1, Ec1, 1)
    dst_c1 = dst.reshape(B * nj1, Ec1, 1)
    dst_r1 = dst.reshape(B * nj1, 1, Ec1)
    ea_c1 = ea.reshape(B * nj1, Ec1, 3)
    we1 = jnp.zeros((8, 256), jnp.float32).at[:3].set(p['c1_We'].T)
    att1 = p['c1_att'].reshape(1, 256)
    num1, den1 = _edge_stage(xl1, xr1, src_c1, dst_c1, dst_r1, ea_c1,
                             we1, att1, H=4, C=64, Ec=Ec1)

    w2 = jnp.concatenate([p['c2_Wl'], p['c2_Wr']], axis=0)
    b2 = jnp.concatenate([p['c2_bl'], p['c2_br']])[None, :]
    xl2, xr2 = _dense2(num1.reshape(B * N, 256), den1.reshape(B * N, 4),
                       p['c1_bias'][None, :], w2.T.astype(bf), b2)
    xl2 = xl2.reshape(B, N, 1024)
    xr2 = xr2.reshape(B, N, 1024)

    Ec2 = 2048
    nj2 = E // Ec2
    src_c2 = src.reshape(B * nj2, Ec2, 1)
    dst_c2 = dst.reshape(B * nj2, Ec2, 1)
    dst_r2 = dst.reshape(B * nj2, 1, Ec2)
    ea_c2 = ea.reshape(B * nj2, Ec2, 3)
    we2 = jnp.zeros((8, 1024), jnp.float32).at[:3].set(p['c2_We'].T)
    att2 = p['c2_att'].reshape(1, 1024)
    num2, den2 = _edge_stage(xl2, xr2, src_c2, dst_c2, dst_r2, ea_c2,
                             we2, att2, H=4, C=256, Ec=Ec2)

    wd = (
        p['proj_W'].T.astype(bf), p['proj_b'][None, :],
        p['gru_Wih'].T.astype(bf), p['gru_Whh'].T.astype(bf),
        p['gru_bih'][None, :], p['gru_bhh'][None, :],
        p['res_W1'].T.astype(bf), p['res_b1'][None, :],
        p['res_W2'].T.astype(bf), p['res_b2'][None, :],
        p['wh_W1'].T.astype(bf), p['wh_b1'][None, :],
        p['wh_W2'].T.astype(bf), p['wh_b2'][None, :],
        p['ap_W1'].T.astype(bf), p['ap_b1'][None, :],
        p['ap_W2'].T.astype(bf), p['ap_b2'][None, :],
        p['ad_W1'].T.astype(bf), p['ad_b1'][None, :],
        p['ad_W2'].T.astype(bf), p['ad_b2'][None, :],
    )
    hn, r, w, ap, ad = _heads(num2.reshape(B * N, 1024),
                              den2.reshape(B * N, 4),
                              p['c2_bias'][None, :],
                              h.reshape(B * N, 256), wd)
    h_new = hn.reshape(B, N, 256)
    return (h_new,
            r.reshape(B, N, 2),
            w.reshape(B, N, 2),
            ap[:, 0, 0:1],
            ad.reshape(B, N, 1))
